# baseline (speedup 1.0000x reference)
import functools

import jax
import jax.numpy as jnp
from jax import lax
from jax.experimental import pallas as pl
from jax.experimental.pallas import tpu as pltpu
from jax.experimental.pallas import tpu_sc as plsc

N_FEAT = 26
B = 1024
L = 50
CARD = 101
EDIM = 8
D = N_FEAT * EDIM
H = 128
G3 = 3 * H
TOP = 32
ROWS = B * L

_NC = 2
_NS = 16
_NW = _NC * _NS
_SC_TOTAL = ROWS * N_FEAT
_PER_W = _SC_TOTAL // _NW
_CH = 5200
_NCHUNK = _PER_W // _CH


def _sc_gather_kernel(table_ref, idx_ref, out_ref, table_v, idx_v, out_v):
    wid = lax.axis_index("s") * _NC + lax.axis_index("c")
    base = wid * _PER_W
    pltpu.sync_copy(table_ref, table_v)
    lanes = lax.iota(jnp.int32, 16)
    out_lane0 = lanes * EDIM

    def chunk_body(ci, carry):
        off = base + ci * _CH
        pltpu.sync_copy(idx_ref.at[pl.ds(off, _CH)], idx_v)

        def vec_body(m, c2):
            bases = idx_v[pl.ds(m * 16, 16)]
            ebase = bases * EDIM
            opos = out_lane0 + m * (16 * EDIM)
            for c in range(EDIM):
                vals = plsc.load_gather(table_v, [ebase + c])
                plsc.store_scatter(out_v, [opos + c], vals)
            return c2

        lax.fori_loop(0, _CH // 16, vec_body, 0)
        pltpu.sync_copy(out_v, out_ref.at[pl.ds(off * EDIM, _CH * EDIM)])
        return carry

    lax.fori_loop(0, _NCHUNK, chunk_body, 0)


def _proj_kernel(x_ref, wx_ref, bx_ref, gi_ref):
    gi_ref[...] = jnp.dot(x_ref[...], wx_ref[...],
                          preferred_element_type=jnp.float32) + bx_ref[0]


def _gru_kernel(gif_ref, gib_ref, whhf_ref, bhhf_ref, whhb_ref, bhhb_ref,
                wc_ref, bc_ref, wh_ref, bh_ref, out_ref,
                hf, hb, mxf, mxb, smf, smb):
    l = pl.program_id(0)

    @pl.when(l == 0)
    def _init():
        zeros = jnp.zeros((B, H), dtype=jnp.float32)
        neg = jnp.full((B, H), -1e30, dtype=jnp.float32)
        hf[...] = zeros
        hb[...] = zeros
        smf[...] = zeros
        smb[...] = zeros
        mxf[...] = neg
        mxb[...] = neg

    def step(gi, h, whhT_ref, bhh_ref):
        gh = jnp.dot(h, whhT_ref[...], preferred_element_type=jnp.float32) + bhh_ref[0]
        r = jax.nn.sigmoid(gi[:, :H] + gh[:, :H])
        z = jax.nn.sigmoid(gi[:, H:2 * H] + gh[:, H:2 * H])
        n = jnp.tanh(gi[:, 2 * H:] + r * gh[:, 2 * H:])
        return (1.0 - z) * n + z * h

    hf_new = step(gif_ref[...], hf[...], whhf_ref, bhhf_ref)
    hb_new = step(gib_ref[...], hb[...], whhb_ref, bhhb_ref)
    hf[...] = hf_new
    hb[...] = hb_new
    mxf[...] = jnp.maximum(mxf[...], hf_new)
    mxb[...] = jnp.maximum(mxb[...], hb_new)
    smf[...] = smf[...] + hf_new
    smb[...] = smb[...] + hb_new

    @pl.when(l == L - 1)
    def _head():
        inv_l = 1.0 / L
        combined = jnp.concatenate(
            [hf[...], hb[...], mxf[...], mxb[...], smf[...] * inv_l, smb[...] * inv_l],
            axis=1)
        act = jax.nn.relu(
            jnp.dot(combined, wc_ref[...], preferred_element_type=jnp.float32)
            + bc_ref[0])
        out_ref[...] = jnp.sum(act * wh_ref[0][None, :], axis=1, keepdims=True) + bh_ref[0]


def kernel(features, emb, Wih_f, Whh_f, bih_f, bhh_f, Wih_b, Whh_b, bih_b, bhh_b,
           Wc, bc, Wh, bh):
    feat3 = jnp.transpose(features, (2, 1, 0)).reshape(ROWS, N_FEAT)
    idx_flat = (feat3 + CARD * jnp.arange(N_FEAT, dtype=jnp.int32)[None, :]
                ).reshape(_SC_TOTAL)
    emb_flat = emb.reshape(N_FEAT * CARD * EDIM)
    Wx = jnp.concatenate([Wih_f, Wih_b], axis=0).T
    bx = jnp.concatenate([bih_f, bih_b]).reshape(1, 2 * G3)
    WhhfT = Whh_f.T
    WhhbT = Whh_b.T
    bhhf2 = bhh_f.reshape(1, G3)
    bhhb2 = bhh_b.reshape(1, G3)
    WcT = Wc.T
    bc2 = bc.reshape(1, TOP)
    bh2 = bh.reshape(1, 1)

    gather = functools.partial(
        pl.kernel,
        mesh=plsc.VectorSubcoreMesh(core_axis_name="c", subcore_axis_name="s"),
        out_type=jax.ShapeDtypeStruct((_SC_TOTAL * EDIM,), jnp.float32),
        scratch_types=[
            pltpu.VMEM((N_FEAT * CARD * EDIM,), jnp.float32),
            pltpu.VMEM((_CH,), jnp.int32),
            pltpu.VMEM((_CH * EDIM,), jnp.float32),
        ],
        compiler_params=pltpu.CompilerParams(needs_layout_passes=False),
    )(_sc_gather_kernel)
    x = gather(emb_flat, idx_flat).reshape(ROWS, D)

    RB = 3200
    gi = pl.pallas_call(
        _proj_kernel,
        grid=(ROWS // RB,),
        in_specs=[
            pl.BlockSpec((RB, D), lambda i: (i, 0)),
            pl.BlockSpec((D, 2 * G3), lambda i: (0, 0)),
            pl.BlockSpec((1, 2 * G3), lambda i: (0, 0)),
        ],
        out_specs=pl.BlockSpec((RB, 2 * G3), lambda i: (i, 0)),
        out_shape=jax.ShapeDtypeStruct((ROWS, 2 * G3), jnp.float32),
    )(x, Wx, bx)

    out = pl.pallas_call(
        _gru_kernel,
        grid=(L,),
        in_specs=[
            pl.BlockSpec((B, G3), lambda l: (l, 0)),
            pl.BlockSpec((B, G3), lambda l: (L - 1 - l, 1)),
            pl.BlockSpec((H, G3), lambda l: (0, 0)),
            pl.BlockSpec((1, G3), lambda l: (0, 0)),
            pl.BlockSpec((H, G3), lambda l: (0, 0)),
            pl.BlockSpec((1, G3), lambda l: (0, 0)),
            pl.BlockSpec((6 * H, TOP), lambda l: (0, 0)),
            pl.BlockSpec((1, TOP), lambda l: (0, 0)),
            pl.BlockSpec((1, TOP), lambda l: (0, 0)),
            pl.BlockSpec((1, 1), lambda l: (0, 0)),
        ],
        out_specs=pl.BlockSpec((B, 1), lambda l: (0, 0)),
        out_shape=jax.ShapeDtypeStruct((B, 1), jnp.float32),
        scratch_shapes=[pltpu.VMEM((B, H), jnp.float32)] * 6,
        compiler_params=pltpu.CompilerParams(
            dimension_semantics=("arbitrary",)),
    )(gi, gi, WhhfT, bhhf2, WhhbT, bhhb2, WcT, bc2, Wh, bh2)
    return out


# speedup vs baseline: 1.1289x; 1.1289x over previous
import functools

import jax
import jax.numpy as jnp
from jax import lax
from jax.experimental import pallas as pl
from jax.experimental.pallas import tpu as pltpu
from jax.experimental.pallas import tpu_sc as plsc

N_FEAT = 26
B = 1024
L = 50
CARD = 101
EDIM = 8
D = N_FEAT * EDIM
H = 128
G3 = 3 * H
TOP = 32
ROWS = B * L

_NC = 2
_NS = 16
_NW = _NC * _NS
_SC_TOTAL = ROWS * N_FEAT
_PER_W = _SC_TOTAL // _NW
_CH = 5200
_NCHUNK = _PER_W // _CH


def _sc_gather_kernel(table_ref, idx_ref, out_ref, idx_v, rows_v, sem):
    wid = lax.axis_index("s") * _NC + lax.axis_index("c")
    base = wid * _PER_W

    def chunk_body(ci, carry):
        off = base + ci * _CH
        pltpu.sync_copy(idx_ref.at[pl.ds(off, _CH)], idx_v)
        pltpu.async_copy(table_ref.at[idx_v], rows_v, sem).wait()
        pltpu.sync_copy(rows_v, out_ref.at[pl.ds(off, _CH)])
        return carry

    lax.fori_loop(0, _NCHUNK, chunk_body, 0)


def _proj_kernel(x_ref, wx_ref, bx_ref, gi_ref):
    gi_ref[...] = jnp.dot(x_ref[...], wx_ref[...],
                          preferred_element_type=jnp.float32) + bx_ref[0]


def _gru_kernel(gif_ref, gib_ref, whhf_ref, bhhf_ref, whhb_ref, bhhb_ref,
                wc_ref, bc_ref, wh_ref, bh_ref, out_ref,
                hf, hb, mxf, mxb, smf, smb):
    l = pl.program_id(0)

    @pl.when(l == 0)
    def _init():
        zeros = jnp.zeros((B, H), dtype=jnp.float32)
        neg = jnp.full((B, H), -1e30, dtype=jnp.float32)
        hf[...] = zeros
        hb[...] = zeros
        smf[...] = zeros
        smb[...] = zeros
        mxf[...] = neg
        mxb[...] = neg

    def step(gi, h, whhT_ref, bhh_ref):
        gh = jnp.dot(h, whhT_ref[...], preferred_element_type=jnp.float32) + bhh_ref[0]
        r = jax.nn.sigmoid(gi[:, :H] + gh[:, :H])
        z = jax.nn.sigmoid(gi[:, H:2 * H] + gh[:, H:2 * H])
        n = jnp.tanh(gi[:, 2 * H:] + r * gh[:, 2 * H:])
        return (1.0 - z) * n + z * h

    hf_new = step(gif_ref[...], hf[...], whhf_ref, bhhf_ref)
    hb_new = step(gib_ref[...], hb[...], whhb_ref, bhhb_ref)
    hf[...] = hf_new
    hb[...] = hb_new
    mxf[...] = jnp.maximum(mxf[...], hf_new)
    mxb[...] = jnp.maximum(mxb[...], hb_new)
    smf[...] = smf[...] + hf_new
    smb[...] = smb[...] + hb_new

    @pl.when(l == L - 1)
    def _head():
        inv_l = 1.0 / L
        combined = jnp.concatenate(
            [hf[...], hb[...], mxf[...], mxb[...], smf[...] * inv_l, smb[...] * inv_l],
            axis=1)
        act = jax.nn.relu(
            jnp.dot(combined, wc_ref[...], preferred_element_type=jnp.float32)
            + bc_ref[0])
        out_ref[...] = jnp.sum(act * wh_ref[0][None, :], axis=1, keepdims=True) + bh_ref[0]


def kernel(features, emb, Wih_f, Whh_f, bih_f, bhh_f, Wih_b, Whh_b, bih_b, bhh_b,
           Wc, bc, Wh, bh):
    feat3 = jnp.transpose(features, (2, 1, 0)).reshape(ROWS, N_FEAT)
    idx_flat = (feat3 + CARD * jnp.arange(N_FEAT, dtype=jnp.int32)[None, :]
                ).reshape(_SC_TOTAL)
    emb_flat = emb.reshape(N_FEAT * CARD, EDIM)
    Wx = jnp.concatenate([Wih_f, Wih_b], axis=0).T
    bx = jnp.concatenate([bih_f, bih_b]).reshape(1, 2 * G3)
    WhhfT = Whh_f.T
    WhhbT = Whh_b.T
    bhhf2 = bhh_f.reshape(1, G3)
    bhhb2 = bhh_b.reshape(1, G3)
    WcT = Wc.T
    bc2 = bc.reshape(1, TOP)
    bh2 = bh.reshape(1, 1)

    gather = functools.partial(
        pl.kernel,
        mesh=plsc.VectorSubcoreMesh(core_axis_name="c", subcore_axis_name="s"),
        out_type=jax.ShapeDtypeStruct((_SC_TOTAL, EDIM), jnp.float32),
        scratch_types=[
            pltpu.VMEM((_CH,), jnp.int32),
            pltpu.VMEM((_CH, EDIM), jnp.float32),
            pltpu.SemaphoreType.DMA,
        ],
        compiler_params=pltpu.CompilerParams(
            needs_layout_passes=False, use_tc_tiling_on_sc=False),
    )(_sc_gather_kernel)
    x = gather(emb_flat, idx_flat).reshape(ROWS, D)

    RB = 3200
    gi = pl.pallas_call(
        _proj_kernel,
        grid=(ROWS // RB,),
        in_specs=[
            pl.BlockSpec((RB, D), lambda i: (i, 0)),
            pl.BlockSpec((D, 2 * G3), lambda i: (0, 0)),
            pl.BlockSpec((1, 2 * G3), lambda i: (0, 0)),
        ],
        out_specs=pl.BlockSpec((RB, 2 * G3), lambda i: (i, 0)),
        out_shape=jax.ShapeDtypeStruct((ROWS, 2 * G3), jnp.float32),
    )(x, Wx, bx)

    out = pl.pallas_call(
        _gru_kernel,
        grid=(L,),
        in_specs=[
            pl.BlockSpec((B, G3), lambda l: (l, 0)),
            pl.BlockSpec((B, G3), lambda l: (L - 1 - l, 1)),
            pl.BlockSpec((H, G3), lambda l: (0, 0)),
            pl.BlockSpec((1, G3), lambda l: (0, 0)),
            pl.BlockSpec((H, G3), lambda l: (0, 0)),
            pl.BlockSpec((1, G3), lambda l: (0, 0)),
            pl.BlockSpec((6 * H, TOP), lambda l: (0, 0)),
            pl.BlockSpec((1, TOP), lambda l: (0, 0)),
            pl.BlockSpec((1, TOP), lambda l: (0, 0)),
            pl.BlockSpec((1, 1), lambda l: (0, 0)),
        ],
        out_specs=pl.BlockSpec((B, 1), lambda l: (0, 0)),
        out_shape=jax.ShapeDtypeStruct((B, 1), jnp.float32),
        scratch_shapes=[pltpu.VMEM((B, H), jnp.float32)] * 6,
        compiler_params=pltpu.CompilerParams(
            dimension_semantics=("arbitrary",)),
    )(gi, gi, WhhfT, bhhf2, WhhbT, bhhb2, WcT, bc2, Wh, bh2)
    return out


# speedup vs baseline: 1.3247x; 1.1735x over previous
import functools

import jax
import jax.numpy as jnp
from jax import lax
from jax.experimental import pallas as pl
from jax.experimental.pallas import tpu as pltpu
from jax.experimental.pallas import tpu_sc as plsc

N_FEAT = 26
B = 1024
L = 50
CARD = 101
EDIM = 8
D = N_FEAT * EDIM
H = 128
G3 = 3 * H
TOP = 32
ROWS = B * L

_NC = 2
_NS = 16
_NW = _NC * _NS
_SC_TOTAL = ROWS * N_FEAT
_PER_W = _SC_TOTAL // _NW
_CH = 5200
_NCHUNK = _PER_W // _CH


def _sc_gather_kernel(table_ref, idx_ref, out_ref, idx_v, rows_v, sem):
    wid = lax.axis_index("s") * _NC + lax.axis_index("c")
    base = wid * _PER_W

    def chunk_body(ci, carry):
        off = base + ci * _CH
        pltpu.sync_copy(idx_ref.at[pl.ds(off, _CH)], idx_v)
        pltpu.async_copy(table_ref.at[idx_v], rows_v, sem).wait()
        pltpu.sync_copy(rows_v, out_ref.at[pl.ds(off, _CH)])
        return carry

    lax.fori_loop(0, _NCHUNK, chunk_body, 0)


def _gru_kernel(xf_ref, xb_ref, wxf_ref, bxf_ref, wxb_ref, bxb_ref,
                whhf_ref, bhhf_ref, whhb_ref, bhhb_ref,
                wc_ref, bc_ref, wh_ref, bh_ref, out_ref,
                hf, hb, mxf, mxb, smf, smb):
    l = pl.program_id(0)

    @pl.when(l == 0)
    def _init():
        zeros = jnp.zeros((B, H), dtype=jnp.float32)
        neg = jnp.full((B, H), -1e30, dtype=jnp.float32)
        hf[...] = zeros
        hb[...] = zeros
        smf[...] = zeros
        smb[...] = zeros
        mxf[...] = neg
        mxb[...] = neg

    def step(x_blk, wx_ref, bx_ref, h, whhT_ref, bhh_ref):
        gi = jnp.dot(x_blk, wx_ref[...], preferred_element_type=jnp.float32) + bx_ref[0]
        gh = jnp.dot(h, whhT_ref[...], preferred_element_type=jnp.float32) + bhh_ref[0]
        r = jax.nn.sigmoid(gi[:, :H] + gh[:, :H])
        z = jax.nn.sigmoid(gi[:, H:2 * H] + gh[:, H:2 * H])
        n = jnp.tanh(gi[:, 2 * H:] + r * gh[:, 2 * H:])
        return (1.0 - z) * n + z * h

    hf_new = step(xf_ref[0], wxf_ref, bxf_ref, hf[...], whhf_ref, bhhf_ref)
    hb_new = step(xb_ref[0], wxb_ref, bxb_ref, hb[...], whhb_ref, bhhb_ref)
    hf[...] = hf_new
    hb[...] = hb_new
    mxf[...] = jnp.maximum(mxf[...], hf_new)
    mxb[...] = jnp.maximum(mxb[...], hb_new)
    smf[...] = smf[...] + hf_new
    smb[...] = smb[...] + hb_new

    @pl.when(l == L - 1)
    def _head():
        inv_l = 1.0 / L
        combined = jnp.concatenate(
            [hf[...], hb[...], mxf[...], mxb[...], smf[...] * inv_l, smb[...] * inv_l],
            axis=1)
        act = jax.nn.relu(
            jnp.dot(combined, wc_ref[...], preferred_element_type=jnp.float32)
            + bc_ref[0])
        out_ref[...] = jnp.sum(act * wh_ref[0][None, :], axis=1, keepdims=True) + bh_ref[0]


def kernel(features, emb, Wih_f, Whh_f, bih_f, bhh_f, Wih_b, Whh_b, bih_b, bhh_b,
           Wc, bc, Wh, bh):
    feat3 = jnp.transpose(features, (2, 1, 0)).reshape(ROWS, N_FEAT)
    idx_flat = (feat3 + CARD * jnp.arange(N_FEAT, dtype=jnp.int32)[None, :]
                ).reshape(_SC_TOTAL)
    emb_flat = emb.reshape(N_FEAT * CARD, EDIM)
    Wxf = Wih_f.T
    Wxb = Wih_b.T
    bxf = bih_f.reshape(1, G3)
    bxb = bih_b.reshape(1, G3)
    WhhfT = Whh_f.T
    WhhbT = Whh_b.T
    bhhf2 = bhh_f.reshape(1, G3)
    bhhb2 = bhh_b.reshape(1, G3)
    WcT = Wc.T
    bc2 = bc.reshape(1, TOP)
    bh2 = bh.reshape(1, 1)

    gather = functools.partial(
        pl.kernel,
        mesh=plsc.VectorSubcoreMesh(core_axis_name="c", subcore_axis_name="s"),
        out_type=jax.ShapeDtypeStruct((_SC_TOTAL, EDIM), jnp.float32),
        scratch_types=[
            pltpu.VMEM((_CH,), jnp.int32),
            pltpu.VMEM((_CH, EDIM), jnp.float32),
            pltpu.SemaphoreType.DMA,
        ],
        compiler_params=pltpu.CompilerParams(
            needs_layout_passes=False, use_tc_tiling_on_sc=False),
    )(_sc_gather_kernel)
    x = gather(emb_flat, idx_flat).reshape(L, B, D)

    out = pl.pallas_call(
        _gru_kernel,
        grid=(L,),
        in_specs=[
            pl.BlockSpec((1, B, D), lambda l: (l, 0, 0)),
            pl.BlockSpec((1, B, D), lambda l: (L - 1 - l, 0, 0)),
            pl.BlockSpec((D, G3), lambda l: (0, 0)),
            pl.BlockSpec((1, G3), lambda l: (0, 0)),
            pl.BlockSpec((D, G3), lambda l: (0, 0)),
            pl.BlockSpec((1, G3), lambda l: (0, 0)),
            pl.BlockSpec((H, G3), lambda l: (0, 0)),
            pl.BlockSpec((1, G3), lambda l: (0, 0)),
            pl.BlockSpec((H, G3), lambda l: (0, 0)),
            pl.BlockSpec((1, G3), lambda l: (0, 0)),
            pl.BlockSpec((6 * H, TOP), lambda l: (0, 0)),
            pl.BlockSpec((1, TOP), lambda l: (0, 0)),
            pl.BlockSpec((1, TOP), lambda l: (0, 0)),
            pl.BlockSpec((1, 1), lambda l: (0, 0)),
        ],
        out_specs=pl.BlockSpec((B, 1), lambda l: (0, 0)),
        out_shape=jax.ShapeDtypeStruct((B, 1), jnp.float32),
        scratch_shapes=[pltpu.VMEM((B, H), jnp.float32)] * 6,
        compiler_params=pltpu.CompilerParams(
            dimension_semantics=("arbitrary",)),
    )(x, x, Wxf, bxf, Wxb, bxb, WhhfT, bhhf2, WhhbT, bhhb2, WcT, bc2, Wh, bh2)
    return out
